# baseline (speedup 1.0000x reference)
import functools

import jax
import jax.numpy as jnp
from jax import lax
from jax.experimental import pallas as pl
from jax.experimental.pallas import tpu as pltpu
from jax.experimental.pallas import tpu_sc as plsc

VOCAB = 90
D = 128
BATCH = 16384
HIST = 200
N = BATCH * HIST
NUM_CORES = 2
NUM_SUBCORES = 16
NW = NUM_CORES * NUM_SUBCORES
PER_W = N // NW
CHUNK = 200
NCHUNK = PER_W // CHUNK
NBUF = 4

assert PER_W * NW == N
assert NCHUNK * CHUNK == PER_W
assert NCHUNK % NBUF == 0
assert CHUNK % 8 == 0 and PER_W % 8 == 0


def _build_kernel():
  mesh = plsc.VectorSubcoreMesh(core_axis_name="c", subcore_axis_name="s")

  @functools.partial(
      pl.kernel,
      mesh=mesh,
      out_type=jax.ShapeDtypeStruct((N, D), jnp.float32),
      scratch_types=(
          [pltpu.VMEM_SHARED((VOCAB, D), jnp.float32)]
          + [pltpu.VMEM((CHUNK,), jnp.int32) for _ in range(NBUF)]
          + [pltpu.VMEM((CHUNK, D), jnp.float32) for _ in range(NBUF)]
          + [pltpu.SemaphoreType.DMA for _ in range(3 * NBUF)]
      ),
  )
  def emb_kernel(idx_hbm, table_hbm, out_hbm, shared_tab, *bufs):
    idx_v = bufs[0:NBUF]
    rows_v = bufs[NBUF:2 * NBUF]
    gsem = bufs[2 * NBUF:3 * NBUF]
    osem = bufs[3 * NBUF:4 * NBUF]
    isem = bufs[4 * NBUF:5 * NBUF]

    sid = lax.axis_index("s")
    wid = sid * NUM_CORES + lax.axis_index("c")
    base = wid * PER_W

    @pl.when(sid == 0)
    def _stage():
      pltpu.sync_copy(table_hbm, shared_tab)

    plsc.subcore_barrier()

    def idx_src(i):
      return idx_hbm.at[pl.ds(base + i * CHUNK, CHUNK)]

    def out_dst(i):
      return out_hbm.at[pl.ds(base + i * CHUNK, CHUNK)]

    pltpu.async_copy(idx_src(0), idx_v[0], isem[0])
    pltpu.async_copy(idx_src(1), idx_v[1], isem[1])

    def body(g, carry):
      for b in range(NBUF):
        i = NBUF * g + b
        pb = (b - 1) % NBUF

        @pl.when(g >= 1)
        def _reclaim():
          pltpu.make_async_copy(rows_v[b], out_dst(i), osem[b]).wait()

        pltpu.make_async_copy(idx_src(i), idx_v[b], isem[b]).wait()
        pltpu.async_copy(shared_tab.at[idx_v[b]], rows_v[b], gsem[b])

        def _advance():
          pltpu.make_async_copy(
              shared_tab.at[idx_v[pb]], rows_v[pb], gsem[pb]).wait()
          nxt = jnp.minimum(i + 1, NCHUNK - 1)
          pltpu.async_copy(idx_src(nxt), idx_v[(b + 1) % NBUF],
                           isem[(b + 1) % NBUF])
          pltpu.async_copy(rows_v[pb], out_dst(i - 1), osem[pb])

        if b == 0:
          pl.when(g >= 1)(_advance)
        else:
          _advance()
      return carry

    lax.fori_loop(0, NCHUNK // NBUF, body, 0)

    last = NCHUNK - 1
    lb = last % NBUF
    pltpu.make_async_copy(shared_tab.at[idx_v[lb]], rows_v[lb],
                          gsem[lb]).wait()
    pltpu.async_copy(rows_v[lb], out_dst(last), osem[lb])
    pltpu.make_async_copy(idx_src(last), idx_v[NCHUNK % NBUF],
                          isem[NCHUNK % NBUF]).wait()
    for b in range(NBUF):
      pltpu.make_async_copy(rows_v[b], out_dst(0), osem[b]).wait()

  return emb_kernel


_EMB_KERNEL = _build_kernel()


@jax.jit
def kernel(note, table):
  flat = note.reshape(-1)
  out = _EMB_KERNEL(flat, table)
  return out.reshape(BATCH, HIST, D)


# speedup vs baseline: 1.0081x; 1.0081x over previous
import functools

import jax
import jax.numpy as jnp
from jax import lax
from jax.experimental import pallas as pl
from jax.experimental.pallas import tpu as pltpu
from jax.experimental.pallas import tpu_sc as plsc

VOCAB = 90
D = 128
BATCH = 16384
HIST = 200
N = BATCH * HIST
NUM_CORES = 2
NUM_SUBCORES = 16
NW = NUM_CORES * NUM_SUBCORES
PER_W = N // NW
CHUNK = 400
NCHUNK = PER_W // CHUNK
NBUF = 2

assert PER_W * NW == N
assert NCHUNK * CHUNK == PER_W
assert NCHUNK % NBUF == 0
assert CHUNK % 8 == 0 and PER_W % 8 == 0


def _build_kernel():
  mesh = plsc.VectorSubcoreMesh(core_axis_name="c", subcore_axis_name="s")

  @functools.partial(
      pl.kernel,
      mesh=mesh,
      out_type=jax.ShapeDtypeStruct((N, D), jnp.float32),
      scratch_types=(
          [pltpu.VMEM_SHARED((VOCAB, D), jnp.float32)]
          + [pltpu.VMEM((CHUNK,), jnp.int32) for _ in range(NBUF)]
          + [pltpu.VMEM((CHUNK, D), jnp.float32) for _ in range(NBUF)]
          + [pltpu.SemaphoreType.DMA for _ in range(3 * NBUF)]
      ),
  )
  def emb_kernel(idx_hbm, table_hbm, out_hbm, shared_tab, *bufs):
    idx_v = bufs[0:NBUF]
    rows_v = bufs[NBUF:2 * NBUF]
    gsem = bufs[2 * NBUF:3 * NBUF]
    osem = bufs[3 * NBUF:4 * NBUF]
    isem = bufs[4 * NBUF:5 * NBUF]

    sid = lax.axis_index("s")
    wid = sid * NUM_CORES + lax.axis_index("c")
    base = wid * PER_W

    @pl.when(sid == 0)
    def _stage():
      pltpu.sync_copy(table_hbm, shared_tab)

    plsc.subcore_barrier()

    def idx_src(i):
      return idx_hbm.at[pl.ds(base + i * CHUNK, CHUNK)]

    def out_dst(i):
      return out_hbm.at[pl.ds(base + i * CHUNK, CHUNK)]

    pltpu.async_copy(idx_src(0), idx_v[0], isem[0])
    pltpu.async_copy(idx_src(1), idx_v[1], isem[1])

    def body(g, carry):
      for b in range(NBUF):
        i = NBUF * g + b
        pb = (b - 1) % NBUF

        @pl.when(g >= 1)
        def _reclaim():
          pltpu.make_async_copy(rows_v[b], out_dst(i), osem[b]).wait()

        pltpu.make_async_copy(idx_src(i), idx_v[b], isem[b]).wait()
        pltpu.async_copy(shared_tab.at[idx_v[b]], rows_v[b], gsem[b])

        def _advance():
          pltpu.make_async_copy(
              shared_tab.at[idx_v[pb]], rows_v[pb], gsem[pb]).wait()
          nxt = jnp.minimum(i + 1, NCHUNK - 1)
          pltpu.async_copy(idx_src(nxt), idx_v[(b + 1) % NBUF],
                           isem[(b + 1) % NBUF])
          pltpu.async_copy(rows_v[pb], out_dst(i - 1), osem[pb])

        if b == 0:
          pl.when(g >= 1)(_advance)
        else:
          _advance()
      return carry

    lax.fori_loop(0, NCHUNK // NBUF, body, 0)

    last = NCHUNK - 1
    lb = last % NBUF
    pltpu.make_async_copy(shared_tab.at[idx_v[lb]], rows_v[lb],
                          gsem[lb]).wait()
    pltpu.async_copy(rows_v[lb], out_dst(last), osem[lb])
    pltpu.make_async_copy(idx_src(last), idx_v[NCHUNK % NBUF],
                          isem[NCHUNK % NBUF]).wait()
    for b in range(NBUF):
      pltpu.make_async_copy(rows_v[b], out_dst(0), osem[b]).wait()

  return emb_kernel


_EMB_KERNEL = _build_kernel()


@jax.jit
def kernel(note, table):
  flat = note.reshape(-1)
  out = _EMB_KERNEL(flat, table)
  return out.reshape(BATCH, HIST, D)


# speedup vs baseline: 1.0155x; 1.0074x over previous
import functools

import jax
import jax.numpy as jnp
from jax import lax
from jax.experimental import pallas as pl
from jax.experimental.pallas import tpu as pltpu
from jax.experimental.pallas import tpu_sc as plsc

VOCAB = 90
D = 128
BATCH = 16384
HIST = 200
N = BATCH * HIST
NUM_CORES = 2
NUM_SUBCORES = 16
NW = NUM_CORES * NUM_SUBCORES
PER_W = N // NW
CHUNK = 320
NCHUNK = PER_W // CHUNK
NBUF = 2

assert PER_W * NW == N
assert NCHUNK * CHUNK == PER_W
assert NCHUNK % NBUF == 0
assert CHUNK % 8 == 0 and PER_W % 8 == 0


def _build_kernel():
  mesh = plsc.VectorSubcoreMesh(core_axis_name="c", subcore_axis_name="s")

  @functools.partial(
      pl.kernel,
      mesh=mesh,
      out_type=jax.ShapeDtypeStruct((N, D), jnp.float32),
      scratch_types=(
          [pltpu.VMEM_SHARED((VOCAB, D), jnp.float32)]
          + [pltpu.VMEM((CHUNK,), jnp.int32) for _ in range(NBUF)]
          + [pltpu.VMEM((CHUNK, D), jnp.float32) for _ in range(NBUF)]
          + [pltpu.SemaphoreType.DMA for _ in range(3 * NBUF)]
      ),
  )
  def emb_kernel(idx_hbm, table_hbm, out_hbm, shared_tab, *bufs):
    idx_v = bufs[0:NBUF]
    rows_v = bufs[NBUF:2 * NBUF]
    gsem = bufs[2 * NBUF:3 * NBUF]
    osem = bufs[3 * NBUF:4 * NBUF]
    isem = bufs[4 * NBUF:5 * NBUF]

    sid = lax.axis_index("s")
    wid = sid * NUM_CORES + lax.axis_index("c")
    base = wid * PER_W

    @pl.when(sid == 0)
    def _stage():
      pltpu.sync_copy(table_hbm, shared_tab)

    plsc.subcore_barrier()

    def idx_src(i):
      return idx_hbm.at[pl.ds(base + i * CHUNK, CHUNK)]

    def out_dst(i):
      return out_hbm.at[pl.ds(base + i * CHUNK, CHUNK)]

    pltpu.async_copy(idx_src(0), idx_v[0], isem[0])
    pltpu.async_copy(idx_src(1), idx_v[1], isem[1])

    def body(g, carry):
      for b in range(NBUF):
        i = NBUF * g + b
        pb = (b - 1) % NBUF

        @pl.when(g >= 1)
        def _reclaim():
          pltpu.make_async_copy(rows_v[b], out_dst(i), osem[b]).wait()

        pltpu.make_async_copy(idx_src(i), idx_v[b], isem[b]).wait()
        pltpu.async_copy(shared_tab.at[idx_v[b]], rows_v[b], gsem[b])

        def _advance():
          pltpu.make_async_copy(
              shared_tab.at[idx_v[pb]], rows_v[pb], gsem[pb]).wait()
          nxt = jnp.minimum(i + 1, NCHUNK - 1)
          pltpu.async_copy(idx_src(nxt), idx_v[(b + 1) % NBUF],
                           isem[(b + 1) % NBUF])
          pltpu.async_copy(rows_v[pb], out_dst(i - 1), osem[pb])

        if b == 0:
          pl.when(g >= 1)(_advance)
        else:
          _advance()
      return carry

    lax.fori_loop(0, NCHUNK // NBUF, body, 0)

    last = NCHUNK - 1
    lb = last % NBUF
    pltpu.make_async_copy(shared_tab.at[idx_v[lb]], rows_v[lb],
                          gsem[lb]).wait()
    pltpu.async_copy(rows_v[lb], out_dst(last), osem[lb])
    pltpu.make_async_copy(idx_src(last), idx_v[NCHUNK % NBUF],
                          isem[NCHUNK % NBUF]).wait()
    for b in range(NBUF):
      pltpu.make_async_copy(rows_v[b], out_dst(0), osem[b]).wait()

  return emb_kernel


_EMB_KERNEL = _build_kernel()


@jax.jit
def kernel(note, table):
  flat = note.reshape(-1)
  out = _EMB_KERNEL(flat, table)
  return out.reshape(BATCH, HIST, D)


# speedup vs baseline: 1.0209x; 1.0053x over previous
import functools

import jax
import jax.numpy as jnp
from jax import lax
from jax.experimental import pallas as pl
from jax.experimental.pallas import tpu as pltpu
from jax.experimental.pallas import tpu_sc as plsc

VOCAB = 90
D = 128
BATCH = 16384
HIST = 200
N = BATCH * HIST
NUM_CORES = 2
NUM_SUBCORES = 16
NW = NUM_CORES * NUM_SUBCORES
PER_W = N // NW
CHUNK = 256
NCHUNK = PER_W // CHUNK
NBUF = 2

assert PER_W * NW == N
assert NCHUNK * CHUNK == PER_W
assert NCHUNK % NBUF == 0
assert CHUNK % 8 == 0 and PER_W % 8 == 0


def _build_kernel():
  mesh = plsc.VectorSubcoreMesh(core_axis_name="c", subcore_axis_name="s")

  @functools.partial(
      pl.kernel,
      mesh=mesh,
      out_type=jax.ShapeDtypeStruct((N, D), jnp.float32),
      scratch_types=(
          [pltpu.VMEM_SHARED((VOCAB, D), jnp.float32)]
          + [pltpu.VMEM((CHUNK,), jnp.int32) for _ in range(NBUF)]
          + [pltpu.VMEM((CHUNK, D), jnp.float32) for _ in range(NBUF)]
          + [pltpu.SemaphoreType.DMA for _ in range(3 * NBUF)]
      ),
  )
  def emb_kernel(idx_hbm, table_hbm, out_hbm, shared_tab, *bufs):
    idx_v = bufs[0:NBUF]
    rows_v = bufs[NBUF:2 * NBUF]
    gsem = bufs[2 * NBUF:3 * NBUF]
    osem = bufs[3 * NBUF:4 * NBUF]
    isem = bufs[4 * NBUF:5 * NBUF]

    sid = lax.axis_index("s")
    wid = sid * NUM_CORES + lax.axis_index("c")
    base = wid * PER_W

    @pl.when(sid == 0)
    def _stage():
      pltpu.sync_copy(table_hbm, shared_tab)

    plsc.subcore_barrier()

    def idx_src(i):
      return idx_hbm.at[pl.ds(base + i * CHUNK, CHUNK)]

    def out_dst(i):
      return out_hbm.at[pl.ds(base + i * CHUNK, CHUNK)]

    pltpu.async_copy(idx_src(0), idx_v[0], isem[0])
    pltpu.async_copy(idx_src(1), idx_v[1], isem[1])

    def body(g, carry):
      for b in range(NBUF):
        i = NBUF * g + b
        pb = (b - 1) % NBUF

        @pl.when(g >= 1)
        def _reclaim():
          pltpu.make_async_copy(rows_v[b], out_dst(i), osem[b]).wait()

        pltpu.make_async_copy(idx_src(i), idx_v[b], isem[b]).wait()
        pltpu.async_copy(shared_tab.at[idx_v[b]], rows_v[b], gsem[b])

        def _advance():
          pltpu.make_async_copy(
              shared_tab.at[idx_v[pb]], rows_v[pb], gsem[pb]).wait()
          nxt = jnp.minimum(i + 1, NCHUNK - 1)
          pltpu.async_copy(idx_src(nxt), idx_v[(b + 1) % NBUF],
                           isem[(b + 1) % NBUF])
          pltpu.async_copy(rows_v[pb], out_dst(i - 1), osem[pb])

        if b == 0:
          pl.when(g >= 1)(_advance)
        else:
          _advance()
      return carry

    lax.fori_loop(0, NCHUNK // NBUF, body, 0)

    last = NCHUNK - 1
    lb = last % NBUF
    pltpu.make_async_copy(shared_tab.at[idx_v[lb]], rows_v[lb],
                          gsem[lb]).wait()
    pltpu.async_copy(rows_v[lb], out_dst(last), osem[lb])
    pltpu.make_async_copy(idx_src(last), idx_v[NCHUNK % NBUF],
                          isem[NCHUNK % NBUF]).wait()
    for b in range(NBUF):
      pltpu.make_async_copy(rows_v[b], out_dst(0), osem[b]).wait()

  return emb_kernel


_EMB_KERNEL = _build_kernel()


@jax.jit
def kernel(note, table):
  flat = note.reshape(-1)
  out = _EMB_KERNEL(flat, table)
  return out.reshape(BATCH, HIST, D)
